# Initial kernel scaffold; baseline (speedup 1.0000x reference)
import jax
import jax.numpy as jnp
from jax.experimental import pallas as pl


def kernel(pointcloud, w1, g1, b1, w2, g2, b2, w3, g3, b3, pw, pb, ln_g, ln_b):
    raise NotImplementedError("write your pallas kernel here")


# speedup vs baseline: 3.0514x; 3.0514x over previous
import functools

import jax
import jax.numpy as jnp
from jax.experimental import pallas as pl
from jax.experimental.pallas import tpu as pltpu


def _fps_body(xyz_ref, out_ref, *, npoint):
    x0 = xyz_ref[0]
    x1 = xyz_ref[1]
    x2 = xyz_ref[2]
    b, n = x0.shape
    col = jax.lax.broadcasted_iota(jnp.int32, (b, n), 1)

    def body(i, carry):
        dists, far = carry
        onehot = col == far
        c0 = jnp.sum(jnp.where(onehot, x0, 0.0), axis=1, keepdims=True)
        c1 = jnp.sum(jnp.where(onehot, x1, 0.0), axis=1, keepdims=True)
        c2 = jnp.sum(jnp.where(onehot, x2, 0.0), axis=1, keepdims=True)
        out_ref[pl.ds(i, 1)] = jnp.concatenate([c0, c1, c2], axis=1)[None]
        d = (x0 - c0) ** 2 + (x1 - c1) ** 2 + (x2 - c2) ** 2
        dists = jnp.minimum(dists, d)
        far = jnp.argmax(dists, axis=1, keepdims=True).astype(jnp.int32)
        return dists, far

    dists0 = jnp.full((b, n), 1e10, dtype=jnp.float32)
    far0 = jnp.zeros((b, 1), dtype=jnp.int32)
    jax.lax.fori_loop(0, npoint, body, (dists0, far0))


def _fps(xyzT, npoint):
    _, b, n = xyzT.shape
    out = pl.pallas_call(
        functools.partial(_fps_body, npoint=npoint),
        out_shape=jax.ShapeDtypeStruct((npoint, b, 3), jnp.float32),
    )(xyzT)
    return jnp.transpose(out, (1, 0, 2))


def _cumsum_lanes(x):
    r, n = x.shape
    k = 1
    while k < n:
        shifted = jnp.concatenate(
            [jnp.zeros((r, k), x.dtype), x[:, : n - k]], axis=1)
        x = x + shifted
        k *= 2
    return x


def _bq_body(xyzT_ref, cat_ref, nxyz_ref, out_ref, *, radius2, nsample, sblk):
    x0 = xyzT_ref[0, 0:1, :]
    x1 = xyzT_ref[0, 1:2, :]
    x2 = xyzT_ref[0, 2:3, :]
    c = nxyz_ref[0]
    d2 = ((c[:, 0:1] - x0) ** 2 + (c[:, 1:2] - x1) ** 2
          + (c[:, 2:3] - x2) ** 2)
    mask = d2 < radius2
    pos = _cumsum_lanes(mask.astype(jnp.int32))
    n = pos.shape[1]
    cat = cat_ref[0]
    ccols = cat.shape[1]
    jcol = jax.lax.broadcasted_iota(jnp.int32, (nsample, 1), 0)
    for s in range(sblk):
        pos_s = pos[s : s + 1, :]
        eq = jnp.logical_and(pos_s == jcol + 1, mask[s : s + 1, :])
        g = jnp.dot(eq.astype(jnp.float32), cat,
                    preferred_element_type=jnp.float32,
                    precision=jax.lax.Precision.HIGHEST)
        valid = jcol < pos_s[:, n - 1 : n]
        g = jnp.where(valid, g, g[0:1, :])
        if ccols == 3:
            sub = c[s : s + 1, :]
        else:
            sub = jnp.concatenate(
                [c[s : s + 1, :], jnp.zeros((1, ccols - 3), jnp.float32)],
                axis=1)
        out_ref[0, s] = g - sub


def _ball_group(xyzT, cat, new_xyz, radius, nsample, sblk):
    b, _, n = xyzT.shape
    ccols = cat.shape[2]
    s_tot = new_xyz.shape[1]
    body = functools.partial(_bq_body, radius2=radius * radius,
                             nsample=nsample, sblk=sblk)
    return pl.pallas_call(
        body,
        grid=(b, s_tot // sblk),
        in_specs=[
            pl.BlockSpec((1, 3, n), lambda i, j: (i, 0, 0)),
            pl.BlockSpec((1, n, ccols), lambda i, j: (i, 0, 0)),
            pl.BlockSpec((1, sblk, 3), lambda i, j: (i, j, 0)),
        ],
        out_specs=pl.BlockSpec((1, sblk, nsample, ccols),
                               lambda i, j: (i, j, 0, 0)),
        out_shape=jax.ShapeDtypeStruct((b, s_tot, nsample, ccols),
                                       jnp.float32),
        compiler_params=pltpu.CompilerParams(
            dimension_semantics=("arbitrary", "arbitrary")),
    )(xyzT, cat, new_xyz)


def _mm_first_body(x_ref, w_ref, y_ref, s_ref, ss_ref):
    y = jnp.dot(x_ref[...], w_ref[...], preferred_element_type=jnp.float32)
    y_ref[...] = y

    @pl.when(pl.program_id(0) == 0)
    def _():
        s_ref[...] = jnp.zeros_like(s_ref)
        ss_ref[...] = jnp.zeros_like(ss_ref)

    s_ref[...] += jnp.sum(y, axis=0, keepdims=True)
    ss_ref[...] += jnp.sum(y * y, axis=0, keepdims=True)


def _mm_norm_body(x_ref, a_ref, c_ref, w_ref, y_ref, s_ref, ss_ref):
    xn = jnp.maximum(x_ref[...] * a_ref[...] + c_ref[...], 0.0)
    y = jnp.dot(xn, w_ref[...], preferred_element_type=jnp.float32)
    y_ref[...] = y

    @pl.when(pl.program_id(0) == 0)
    def _():
        s_ref[...] = jnp.zeros_like(s_ref)
        ss_ref[...] = jnp.zeros_like(ss_ref)

    s_ref[...] += jnp.sum(y, axis=0, keepdims=True)
    ss_ref[...] += jnp.sum(y * y, axis=0, keepdims=True)


def _stats_to_affine(s, ss, m, g, bb):
    mu = s / m
    var = ss / m - mu * mu
    a = (g[None, :] / jnp.sqrt(var + 1e-5))
    return a, bb[None, :] - mu * a


def _mm_layer(x, w, g, bb, ac=None, rblk=4096):
    m, cin = x.shape
    cout = w.shape[1]
    outs = (jax.ShapeDtypeStruct((m, cout), jnp.float32),
            jax.ShapeDtypeStruct((1, cout), jnp.float32),
            jax.ShapeDtypeStruct((1, cout), jnp.float32))
    out_specs = (pl.BlockSpec((rblk, cout), lambda i: (i, 0)),
                 pl.BlockSpec((1, cout), lambda i: (0, 0)),
                 pl.BlockSpec((1, cout), lambda i: (0, 0)))
    if ac is None:
        body = _mm_first_body
        in_specs = [pl.BlockSpec((rblk, cin), lambda i: (i, 0)),
                    pl.BlockSpec((cin, cout), lambda i: (0, 0))]
        args = (x, w)
    else:
        body = _mm_norm_body
        in_specs = [pl.BlockSpec((rblk, cin), lambda i: (i, 0)),
                    pl.BlockSpec((1, cin), lambda i: (0, 0)),
                    pl.BlockSpec((1, cin), lambda i: (0, 0)),
                    pl.BlockSpec((cin, cout), lambda i: (0, 0))]
        args = (x, ac[0], ac[1], w)
    y, s, ss = pl.pallas_call(
        body,
        grid=(m // rblk,),
        in_specs=in_specs,
        out_specs=list(out_specs),
        out_shape=list(outs),
        compiler_params=pltpu.CompilerParams(
            dimension_semantics=("arbitrary",)),
    )(*args)
    a, c = _stats_to_affine(s[0], ss[0], float(m), g, bb)
    return y, (a, c)


def _fin_body(y_ref, a_ref, c_ref, o_ref, *, k):
    x = jnp.maximum(y_ref[...] * a_ref[...] + c_ref[...], 0.0)
    r, ccols = x.shape
    o_ref[...] = jnp.max(x.reshape(r // k, k, ccols), axis=1)


def _maxpool(y, ac, k, rblk=4096):
    m, ccols = y.shape
    return pl.pallas_call(
        functools.partial(_fin_body, k=k),
        grid=(m // rblk,),
        in_specs=[pl.BlockSpec((rblk, ccols), lambda i: (i, 0)),
                  pl.BlockSpec((1, ccols), lambda i: (0, 0)),
                  pl.BlockSpec((1, ccols), lambda i: (0, 0))],
        out_specs=pl.BlockSpec((rblk // k, ccols), lambda i: (i, 0)),
        out_shape=jax.ShapeDtypeStruct((m // k, ccols), jnp.float32),
        compiler_params=pltpu.CompilerParams(
            dimension_semantics=("arbitrary",)),
    )(y, ac[0], ac[1])


def _tail_body(x_ref, w1_ref, g1_ref, b1_ref, w2_ref, g2_ref, b2_ref,
               w3_ref, g3_ref, b3_ref, pw_ref, pb_ref, lng_ref, lnb_ref,
               o_ref, *, b, k):
    def bn_relu(y, g, bb):
        mu = jnp.mean(y, axis=0, keepdims=True)
        var = jnp.mean((y - mu) ** 2, axis=0, keepdims=True)
        return jnp.maximum((y - mu) / jnp.sqrt(var + 1e-5) * g + bb, 0.0)

    x = x_ref[...]
    x = bn_relu(jnp.dot(x, w1_ref[...], preferred_element_type=jnp.float32),
                g1_ref[...], b1_ref[...])
    x = bn_relu(jnp.dot(x, w2_ref[...], preferred_element_type=jnp.float32),
                g2_ref[...], b2_ref[...])
    x = bn_relu(jnp.dot(x, w3_ref[...], preferred_element_type=jnp.float32),
                g3_ref[...], b3_ref[...])
    xm = jnp.max(x.reshape(b, k, x.shape[1]), axis=1)
    yp = jnp.dot(xm, pw_ref[...], preferred_element_type=jnp.float32)
    yp = yp + pb_ref[...]
    mu = jnp.mean(yp, axis=1, keepdims=True)
    var = jnp.mean((yp - mu) ** 2, axis=1, keepdims=True)
    yn = (yp - mu) / jnp.sqrt(var + 1e-5) * lng_ref[...] + lnb_ref[...]
    inv_sqrt2 = 0.7071067811865476
    o_ref[...] = yn * 0.5 * (1.0 + jax.lax.erf(yn * inv_sqrt2))


def _tail(x, w3l, g3l, b3l, pw, pb, ln_g, ln_b, b, k):
    args = (x,
            w3l[0], g3l[0][None, :], b3l[0][None, :],
            w3l[1], g3l[1][None, :], b3l[1][None, :],
            w3l[2], g3l[2][None, :], b3l[2][None, :],
            pw, pb[None, :], ln_g[None, :], ln_b[None, :])
    return pl.pallas_call(
        functools.partial(_tail_body, b=b, k=k),
        out_shape=jax.ShapeDtypeStruct((b, pw.shape[1]), jnp.float32),
    )(*args)


def kernel(pointcloud, w1, g1, b1, w2, g2, b2, w3, g3, b3, pw, pb, ln_g, ln_b):
    b, n, _ = pointcloud.shape

    xyzT = jnp.transpose(pointcloud, (2, 0, 1))
    nx1 = _fps(xyzT, 512)
    xyzTb = jnp.transpose(pointcloud, (0, 2, 1))
    grouped1 = _ball_group(xyzTb, pointcloud, nx1, 0.2, 64, 8)
    x = grouped1.reshape(b * 512 * 64, 3)
    y, ac = _mm_layer(x, w1[0], g1[0], b1[0])
    y, ac = _mm_layer(y, w1[1], g1[1], b1[1], ac)
    y, ac = _mm_layer(y, w1[2], g1[2], b1[2], ac)
    x1 = _maxpool(y, ac, 64).reshape(b, 512, 128)

    nx1T = jnp.transpose(nx1, (2, 0, 1))
    nx2 = _fps(nx1T, 128)
    cat1 = jnp.concatenate([nx1, x1], axis=-1)
    nx1Tb = jnp.transpose(nx1, (0, 2, 1))
    grouped2 = _ball_group(nx1Tb, cat1, nx2, 0.4, 64, 8)
    x = grouped2.reshape(b * 128 * 64, 131)
    y, ac = _mm_layer(x, w2[0], g2[0], b2[0])
    y, ac = _mm_layer(y, w2[1], g2[1], b2[1], ac)
    y, ac = _mm_layer(y, w2[2], g2[2], b2[2], ac)
    x2 = _maxpool(y, ac, 64).reshape(b, 128, 256)

    cat2 = jnp.concatenate([nx2, x2], axis=-1).reshape(b * 128, 259)
    return _tail(cat2, w3, g3, b3, pw, pb, ln_g, ln_b, b, 128)
